# baseline (speedup 1.0000x reference)
import jax
import jax.numpy as jnp
import numpy as np
from jax import lax
from jax.experimental import pallas as pl
from jax.experimental.pallas import tpu as pltpu
from jax.experimental.pallas import tpu_sc as plsc

_FP_DIM = 1024
_D_OUT = 1024
_BATCH = 16384
_LN_EPS = 1e-5
_NEG = 0.01

_NC, _NS = 2, 16
_NW = _NC * _NS
_NSPLIT = 4
_CB = _BATCH // _NSPLIT
_BPW = _CB // _NW
_CH = 32
_NCHUNK = _BPW // _CH


def _gather_body(table_hbm, idx_hbm, out_hbm, idx_v, rows_v, sem0, sem1):
    wid = lax.axis_index("s") * _NC + lax.axis_index("c")
    base = wid * _BPW
    sems = (sem0, sem1)
    pltpu.sync_copy(idx_hbm.at[pl.ds(base, _BPW)], idx_v)
    cps = []
    for c in range(_NCHUNK):
        cp = pltpu.make_async_copy(
            table_hbm.at[idx_v.at[pl.ds(c * _CH, _CH)]],
            rows_v.at[c % 2], sems[c % 2])
        cp.start()
        cps.append(cp)
        if c > 0:
            cps[c - 1].wait()
            pltpu.sync_copy(rows_v.at[(c - 1) % 2],
                            out_hbm.at[pl.ds(base + (c - 1) * _CH, _CH)])
    cps[_NCHUNK - 1].wait()
    pltpu.sync_copy(rows_v.at[(_NCHUNK - 1) % 2],
                    out_hbm.at[pl.ds(base + (_NCHUNK - 1) * _CH, _CH)])


def _sc_gather(table, x):
    mesh = plsc.VectorSubcoreMesh(core_axis_name="c", subcore_axis_name="s")
    return pl.kernel(
        _gather_body,
        out_type=jax.ShapeDtypeStruct((_CB, _FP_DIM), jnp.float32),
        mesh=mesh,
        scratch_types=[
            pltpu.VMEM((_BPW,), jnp.int32),
            pltpu.VMEM((2, _CH, _FP_DIM), jnp.float32),
            pltpu.SemaphoreType.DMA,
            pltpu.SemaphoreType.DMA,
        ],
    )(table, x)


_BM = 1024


def _mlp_body(h_ref, w1_ref, b1_ref, w2_ref, b2_ref, g_ref, bb_ref, o_ref):
    _SLAB = _BM // 4
    for s in range(4):
        rows = pl.ds(s * _SLAB, _SLAB)
        h = h_ref[rows, :].astype(jnp.bfloat16)
        a = jnp.dot(h, w1_ref[...], preferred_element_type=jnp.float32) + b1_ref[...]
        a = jnp.where(a > 0, a, a * _NEG)
        p = jnp.dot(a.astype(jnp.bfloat16), w2_ref[...],
                    preferred_element_type=jnp.float32) + b2_ref[...]
        mu = jnp.mean(p, axis=1, keepdims=True)
        d = p - mu
        var = jnp.mean(d * d, axis=1, keepdims=True)
        o_ref[rows, :] = d * lax.rsqrt(var + _LN_EPS) * g_ref[...] + bb_ref[...]


def _mlp_chain_body(buf_ref, h_ref, w1_ref, b1_ref, w2_ref, b2_ref, g_ref,
                    bb_ref, o_ref):
    del buf_ref
    _mlp_body(h_ref, w1_ref, b1_ref, w2_ref, b2_ref, g_ref, bb_ref, o_ref)


_NB = _CB // _BM


def _mlp_first(g, w1, b1, w2, b2, gg, gb):
    return pl.pallas_call(
        _mlp_body,
        grid=(_NB,),
        in_specs=[
            pl.BlockSpec((_BM, _FP_DIM), lambda i: (i, 0)),
            pl.BlockSpec((_FP_DIM, _D_OUT), lambda i: (0, 0)),
            pl.BlockSpec((1, _D_OUT), lambda i: (0, 0)),
            pl.BlockSpec((_D_OUT, _D_OUT), lambda i: (0, 0)),
            pl.BlockSpec((1, _D_OUT), lambda i: (0, 0)),
            pl.BlockSpec((1, _D_OUT), lambda i: (0, 0)),
            pl.BlockSpec((1, _D_OUT), lambda i: (0, 0)),
        ],
        out_specs=pl.BlockSpec((_BM, _D_OUT), lambda i: (i, 0)),
        out_shape=jax.ShapeDtypeStruct((_BATCH, _D_OUT), jnp.float32),
    )(g, w1, b1, w2, b2, gg, gb)


def _mlp_chain(c, buf, g, w1, b1, w2, b2, gg, gb):
    off = c * _NB
    return pl.pallas_call(
        _mlp_chain_body,
        grid=(_NB,),
        in_specs=[
            pl.BlockSpec(memory_space=pl.ANY),
            pl.BlockSpec((_BM, _FP_DIM), lambda i: (i, 0)),
            pl.BlockSpec((_FP_DIM, _D_OUT), lambda i: (0, 0)),
            pl.BlockSpec((1, _D_OUT), lambda i: (0, 0)),
            pl.BlockSpec((_D_OUT, _D_OUT), lambda i: (0, 0)),
            pl.BlockSpec((1, _D_OUT), lambda i: (0, 0)),
            pl.BlockSpec((1, _D_OUT), lambda i: (0, 0)),
            pl.BlockSpec((1, _D_OUT), lambda i: (0, 0)),
        ],
        out_specs=pl.BlockSpec((_BM, _D_OUT), lambda i: (i + off, 0)),
        out_shape=jax.ShapeDtypeStruct((_BATCH, _D_OUT), jnp.float32),
        input_output_aliases={0: 0},
    )(buf, g, w1, b1, w2, b2, gg, gb)


def kernel(x, table, fc_w, fc_b, proj_w, proj_b, ln_g, ln_b):
    w1 = fc_w.T.astype(jnp.bfloat16)
    w2 = proj_w.T.astype(jnp.bfloat16)
    b1 = fc_b.reshape(1, _D_OUT)
    b2 = proj_b.reshape(1, _D_OUT)
    gg = ln_g.reshape(1, _D_OUT)
    gb = ln_b.reshape(1, _D_OUT)
    gs = [_sc_gather(table, lax.slice(x, (c * _CB,), ((c + 1) * _CB,)))
          for c in range(_NSPLIT)]
    buf = _mlp_first(gs[0], w1, b1, w2, b2, gg, gb)
    for c in range(1, _NSPLIT):
        buf = _mlp_chain(c, buf, gs[c], w1, b1, w2, b2, gg, gb)
    return buf


# speedup vs baseline: 1.0544x; 1.0544x over previous
import jax
import jax.numpy as jnp
import numpy as np
from jax import lax
from jax.experimental import pallas as pl
from jax.experimental.pallas import tpu as pltpu
from jax.experimental.pallas import tpu_sc as plsc

_FP_DIM = 1024
_D_OUT = 1024
_BATCH = 16384
_LN_EPS = 1e-5
_NEG = 0.01

_NC, _NS = 2, 16
_NW = _NC * _NS
_NSPLIT = 4
_CB = _BATCH // _NSPLIT
_BPW = _CB // _NW
_CH = 64
_NCHUNK = _BPW // _CH


def _gather_body(table_hbm, idx_hbm, out_hbm, idx_v, rows_v, sem):
    wid = lax.axis_index("s") * _NC + lax.axis_index("c")
    base = wid * _BPW
    for c in range(_NCHUNK):
        pltpu.sync_copy(idx_hbm.at[pl.ds(base + c * _CH, _CH)], idx_v)
        pltpu.async_copy(table_hbm.at[idx_v], rows_v, sem).wait()
        pltpu.sync_copy(rows_v, out_hbm.at[pl.ds(base + c * _CH, _CH)])


def _sc_gather(table, x):
    mesh = plsc.VectorSubcoreMesh(core_axis_name="c", subcore_axis_name="s")
    return pl.kernel(
        _gather_body,
        out_type=jax.ShapeDtypeStruct((_CB, _FP_DIM), jnp.float32),
        mesh=mesh,
        scratch_types=[
            pltpu.VMEM((_CH,), jnp.int32),
            pltpu.VMEM((_CH, _FP_DIM), jnp.float32),
            pltpu.SemaphoreType.DMA,
        ],
    )(table, x)


_BM = 1024


def _mlp_body(h_ref, w1_ref, b1_ref, w2_ref, b2_ref, g_ref, bb_ref, o_ref):
    _SLAB = _BM // 4
    for s in range(4):
        rows = pl.ds(s * _SLAB, _SLAB)
        h = h_ref[rows, :].astype(jnp.bfloat16)
        a = jnp.dot(h, w1_ref[...], preferred_element_type=jnp.float32) + b1_ref[...]
        a = jnp.where(a > 0, a, a * _NEG)
        p = jnp.dot(a.astype(jnp.bfloat16), w2_ref[...],
                    preferred_element_type=jnp.float32) + b2_ref[...]
        mu = jnp.mean(p, axis=1, keepdims=True)
        d = p - mu
        var = jnp.mean(d * d, axis=1, keepdims=True)
        o_ref[rows, :] = d * lax.rsqrt(var + _LN_EPS) * g_ref[...] + bb_ref[...]


def _mlp_chain_body(buf_ref, h_ref, w1_ref, b1_ref, w2_ref, b2_ref, g_ref,
                    bb_ref, o_ref):
    del buf_ref
    _mlp_body(h_ref, w1_ref, b1_ref, w2_ref, b2_ref, g_ref, bb_ref, o_ref)


_NB = _CB // _BM


def _mlp_first(g, w1, b1, w2, b2, gg, gb):
    return pl.pallas_call(
        _mlp_body,
        grid=(_NB,),
        in_specs=[
            pl.BlockSpec((_BM, _FP_DIM), lambda i: (i, 0)),
            pl.BlockSpec((_FP_DIM, _D_OUT), lambda i: (0, 0)),
            pl.BlockSpec((1, _D_OUT), lambda i: (0, 0)),
            pl.BlockSpec((_D_OUT, _D_OUT), lambda i: (0, 0)),
            pl.BlockSpec((1, _D_OUT), lambda i: (0, 0)),
            pl.BlockSpec((1, _D_OUT), lambda i: (0, 0)),
            pl.BlockSpec((1, _D_OUT), lambda i: (0, 0)),
        ],
        out_specs=pl.BlockSpec((_BM, _D_OUT), lambda i: (i, 0)),
        out_shape=jax.ShapeDtypeStruct((_BATCH, _D_OUT), jnp.float32),
    )(g, w1, b1, w2, b2, gg, gb)


def _mlp_chain(c, buf, g, w1, b1, w2, b2, gg, gb):
    off = c * _NB
    return pl.pallas_call(
        _mlp_chain_body,
        grid=(_NB,),
        in_specs=[
            pl.BlockSpec(memory_space=pl.ANY),
            pl.BlockSpec((_BM, _FP_DIM), lambda i: (i, 0)),
            pl.BlockSpec((_FP_DIM, _D_OUT), lambda i: (0, 0)),
            pl.BlockSpec((1, _D_OUT), lambda i: (0, 0)),
            pl.BlockSpec((_D_OUT, _D_OUT), lambda i: (0, 0)),
            pl.BlockSpec((1, _D_OUT), lambda i: (0, 0)),
            pl.BlockSpec((1, _D_OUT), lambda i: (0, 0)),
            pl.BlockSpec((1, _D_OUT), lambda i: (0, 0)),
        ],
        out_specs=pl.BlockSpec((_BM, _D_OUT), lambda i: (i + off, 0)),
        out_shape=jax.ShapeDtypeStruct((_BATCH, _D_OUT), jnp.float32),
        input_output_aliases={0: 0},
    )(buf, g, w1, b1, w2, b2, gg, gb)


def kernel(x, table, fc_w, fc_b, proj_w, proj_b, ln_g, ln_b):
    w1 = fc_w.T.astype(jnp.bfloat16)
    w2 = proj_w.T.astype(jnp.bfloat16)
    b1 = fc_b.reshape(1, _D_OUT)
    b2 = proj_b.reshape(1, _D_OUT)
    gg = ln_g.reshape(1, _D_OUT)
    gb = ln_b.reshape(1, _D_OUT)
    gs = [_sc_gather(table, lax.slice(x, (c * _CB,), ((c + 1) * _CB,)))
          for c in range(_NSPLIT)]
    buf = _mlp_first(gs[0], w1, b1, w2, b2, gg, gb)
    for c in range(1, _NSPLIT):
        buf = _mlp_chain(c, buf, gs[c], w1, b1, w2, b2, gg, gb)
    return buf


# speedup vs baseline: 1.0628x; 1.0080x over previous
import jax
import jax.numpy as jnp
import numpy as np
from jax import lax
from jax.experimental import pallas as pl
from jax.experimental.pallas import tpu as pltpu
from jax.experimental.pallas import tpu_sc as plsc

_FP_DIM = 1024
_D_OUT = 1024
_BATCH = 16384
_LN_EPS = 1e-5
_NEG = 0.01

_NC, _NS = 2, 16
_NW = _NC * _NS
_NSPLIT = 2
_CB = _BATCH // _NSPLIT
_BPW = _CB // _NW
_CH = 64
_NCHUNK = _BPW // _CH


def _gather_body(table_hbm, idx_hbm, out_hbm, idx_v, rows_v, sem):
    wid = lax.axis_index("s") * _NC + lax.axis_index("c")
    base = wid * _BPW
    for c in range(_NCHUNK):
        pltpu.sync_copy(idx_hbm.at[pl.ds(base + c * _CH, _CH)], idx_v)
        pltpu.async_copy(table_hbm.at[idx_v], rows_v, sem).wait()
        pltpu.sync_copy(rows_v, out_hbm.at[pl.ds(base + c * _CH, _CH)])


def _sc_gather(table, x):
    mesh = plsc.VectorSubcoreMesh(core_axis_name="c", subcore_axis_name="s")
    return pl.kernel(
        _gather_body,
        out_type=jax.ShapeDtypeStruct((_CB, _FP_DIM), jnp.float32),
        mesh=mesh,
        scratch_types=[
            pltpu.VMEM((_CH,), jnp.int32),
            pltpu.VMEM((_CH, _FP_DIM), jnp.float32),
            pltpu.SemaphoreType.DMA,
        ],
    )(table, x)


_BM = 1024


def _mlp_body(h_ref, w1_ref, b1_ref, w2_ref, b2_ref, g_ref, bb_ref, o_ref):
    _SLAB = _BM // 4
    for s in range(4):
        rows = pl.ds(s * _SLAB, _SLAB)
        h = h_ref[rows, :].astype(jnp.bfloat16)
        a = jnp.dot(h, w1_ref[...], preferred_element_type=jnp.float32) + b1_ref[...]
        a = jnp.where(a > 0, a, a * _NEG)
        p = jnp.dot(a.astype(jnp.bfloat16), w2_ref[...],
                    preferred_element_type=jnp.float32) + b2_ref[...]
        mu = jnp.mean(p, axis=1, keepdims=True)
        d = p - mu
        var = jnp.mean(d * d, axis=1, keepdims=True)
        o_ref[rows, :] = d * lax.rsqrt(var + _LN_EPS) * g_ref[...] + bb_ref[...]


def _mlp_chain_body(buf_ref, h_ref, w1_ref, b1_ref, w2_ref, b2_ref, g_ref,
                    bb_ref, o_ref):
    del buf_ref
    _mlp_body(h_ref, w1_ref, b1_ref, w2_ref, b2_ref, g_ref, bb_ref, o_ref)


_NB = _CB // _BM


def _mlp_first(g, w1, b1, w2, b2, gg, gb):
    return pl.pallas_call(
        _mlp_body,
        grid=(_NB,),
        in_specs=[
            pl.BlockSpec((_BM, _FP_DIM), lambda i: (i, 0)),
            pl.BlockSpec((_FP_DIM, _D_OUT), lambda i: (0, 0)),
            pl.BlockSpec((1, _D_OUT), lambda i: (0, 0)),
            pl.BlockSpec((_D_OUT, _D_OUT), lambda i: (0, 0)),
            pl.BlockSpec((1, _D_OUT), lambda i: (0, 0)),
            pl.BlockSpec((1, _D_OUT), lambda i: (0, 0)),
            pl.BlockSpec((1, _D_OUT), lambda i: (0, 0)),
        ],
        out_specs=pl.BlockSpec((_BM, _D_OUT), lambda i: (i, 0)),
        out_shape=jax.ShapeDtypeStruct((_BATCH, _D_OUT), jnp.float32),
    )(g, w1, b1, w2, b2, gg, gb)


def _mlp_chain(c, buf, g, w1, b1, w2, b2, gg, gb):
    off = c * _NB
    return pl.pallas_call(
        _mlp_chain_body,
        grid=(_NB,),
        in_specs=[
            pl.BlockSpec(memory_space=pl.ANY),
            pl.BlockSpec((_BM, _FP_DIM), lambda i: (i, 0)),
            pl.BlockSpec((_FP_DIM, _D_OUT), lambda i: (0, 0)),
            pl.BlockSpec((1, _D_OUT), lambda i: (0, 0)),
            pl.BlockSpec((_D_OUT, _D_OUT), lambda i: (0, 0)),
            pl.BlockSpec((1, _D_OUT), lambda i: (0, 0)),
            pl.BlockSpec((1, _D_OUT), lambda i: (0, 0)),
            pl.BlockSpec((1, _D_OUT), lambda i: (0, 0)),
        ],
        out_specs=pl.BlockSpec((_BM, _D_OUT), lambda i: (i + off, 0)),
        out_shape=jax.ShapeDtypeStruct((_BATCH, _D_OUT), jnp.float32),
        input_output_aliases={0: 0},
    )(buf, g, w1, b1, w2, b2, gg, gb)


def kernel(x, table, fc_w, fc_b, proj_w, proj_b, ln_g, ln_b):
    w1 = fc_w.T.astype(jnp.bfloat16)
    w2 = proj_w.T.astype(jnp.bfloat16)
    b1 = fc_b.reshape(1, _D_OUT)
    b2 = proj_b.reshape(1, _D_OUT)
    gg = ln_g.reshape(1, _D_OUT)
    gb = ln_b.reshape(1, _D_OUT)
    gs = [_sc_gather(table, lax.slice(x, (c * _CB,), ((c + 1) * _CB,)))
          for c in range(_NSPLIT)]
    buf = _mlp_first(gs[0], w1, b1, w2, b2, gg, gb)
    for c in range(1, _NSPLIT):
        buf = _mlp_chain(c, buf, gs[c], w1, b1, w2, b2, gg, gb)
    return buf
